# baseline (speedup 1.0000x reference)
import functools

import jax
import jax.numpy as jnp
from jax import lax
from jax.experimental import pallas as pl
from jax.experimental.pallas import tpu as pltpu
from jax.experimental.pallas import tpu_sc as plsc

N = 10000
E = 320000
DIN = 128
H = 64
DOUT = 32
TW = 128

NC = 2
NS = 16
NW = NC * NS
C = 128
EW = 10240
NCH = EW // C
E_P = NW * EW

BN = 2000
BE = 2048

@functools.cache
def _sc_mesh():
    return plsc.VectorSubcoreMesh(
        core_axis_name="c", subcore_axis_name="s",
        num_cores=NC, num_subcores=NS)


def _tables_body(x_ref, pos_ref, w1_ref, b1_ref, td_ref, ts_ref):
    xb = x_ref[...]
    a = jnp.dot(xb, w1_ref[0:DIN, :], preferred_element_type=jnp.float32)
    b = jnp.dot(xb, w1_ref[DIN:2 * DIN, :], preferred_element_type=jnp.float32)
    p = pos_ref[...]
    z = jnp.zeros((BN, TW - H - 3), jnp.float32)
    td_ref[...] = jnp.concatenate([a + b1_ref[...], p, z], axis=1)
    ts_ref[...] = jnp.concatenate([b, -p, z], axis=1)


def _build_tables(x, pos, W_e1, b1r):
    return pl.pallas_call(
        _tables_body,
        grid=(N // BN,),
        in_specs=[
            pl.BlockSpec((BN, DIN), lambda i: (i, 0)),
            pl.BlockSpec((BN, 3), lambda i: (i, 0)),
            pl.BlockSpec((2 * DIN + 1, H), lambda i: (0, 0)),
            pl.BlockSpec((1, H), lambda i: (0, 0)),
        ],
        out_specs=[
            pl.BlockSpec((BN, TW), lambda i: (i, 0)),
            pl.BlockSpec((BN, TW), lambda i: (i, 0)),
        ],
        out_shape=[
            jax.ShapeDtypeStruct((N, TW), jnp.float32),
            jax.ShapeDtypeStruct((N, TW), jnp.float32),
        ],
    )(x, pos, W_e1, b1r)


NBUF = 4
NCHUNK = E_P // C
K_LIGHT = 24
K_HEAVY = (NCHUNK - NS * K_LIGHT) // NS
LIGHT_CORE = 1


@functools.cache
def _make_gather_kernel():
    @functools.partial(
        pl.kernel,
        out_type=jax.ShapeDtypeStruct((E_P, TW), jnp.float32),
        mesh=_sc_mesh(),
        scratch_types=[
            pltpu.VMEM((K_HEAVY, C), jnp.int32),
            pltpu.VMEM((K_HEAVY, C), jnp.int32),
        ] + [pltpu.VMEM((C, TW), jnp.float32) for _ in range(NBUF)] + [
            pltpu.SemaphoreType.DMA((NBUF,)),
            pltpu.SemaphoreType.DMA((NBUF,)),
        ],
    )
    def _gather_kernel(td_hbm, ts_hbm, idxd_hbm, idxs_hbm, g_hbm,
                       idxd_v, idxs_v, b0, b1, b2, b3, gsem, osem):
        cid = lax.axis_index("c")
        sid = lax.axis_index("s")
        is_light = cid == LIGHT_CORE
        my_k = jnp.where(is_light, K_LIGHT, K_HEAVY)
        cbase = jnp.where(is_light, sid * K_LIGHT,
                          NS * K_LIGHT + sid * K_HEAVY)
        pltpu.sync_copy(idxd_hbm.at[pl.ds(cbase, K_HEAVY)], idxd_v)
        pltpu.sync_copy(idxs_hbm.at[pl.ds(cbase, K_HEAVY)], idxs_v)
        bufs = (b0, b1, b2, b3)

        for b in range(NBUF):
            pltpu.async_copy(td_hbm.at[idxd_v.at[b]], bufs[b], gsem.at[b])

        @pl.loop(0, my_k // NBUF)
        def _grp(g):
            for b in range(NBUF):
                pltpu.make_async_copy(
                    td_hbm.at[idxd_v.at[b]], bufs[b], gsem.at[b]).wait()
                pltpu.async_copy(ts_hbm.at[idxs_v.at[g * NBUF + b]], bufs[b],
                                 gsem.at[b], add=True)
            for b in range(NBUF):
                j = g * NBUF + b
                pltpu.make_async_copy(
                    ts_hbm.at[idxs_v.at[b]], bufs[b], gsem.at[b]).wait()
                pltpu.async_copy(bufs[b],
                                 g_hbm.at[pl.ds((cbase + j) * C, C)],
                                 osem.at[b])
            for b in range(NBUF):
                j = (g + 1) * NBUF + b

                @pl.when(j < my_k)
                def _():
                    pltpu.make_async_copy(
                        bufs[b], g_hbm.at[pl.ds(0, C)], osem.at[b]).wait()
                    pltpu.async_copy(td_hbm.at[idxd_v.at[j]], bufs[b],
                                     gsem.at[b])

        for b in range(NBUF):
            pltpu.make_async_copy(
                bufs[b], g_hbm.at[pl.ds(0, C)], osem.at[b]).wait()

    return _gather_kernel


def _sc_gather(td, ts, dst_c, src_c):
    return _make_gather_kernel()(td, ts, dst_c, src_c)


def _edge_body(g_ref, w2_ref, b2_ref, sel_ref, m_ref):
    i = pl.program_id(0)
    g = g_ref[...]
    pre = (jnp.dot(g, sel_ref[0:TW, :], preferred_element_type=jnp.float32)
           + jnp.dot(g * g, sel_ref[TW:2 * TW, :],
                     preferred_element_type=jnp.float32))
    h1 = pre * jax.nn.sigmoid(pre)
    mm = jnp.dot(h1, w2_ref[...], preferred_element_type=jnp.float32) + b2_ref[...]
    mm = mm * jax.nn.sigmoid(mm)
    rows = i * BE + lax.broadcasted_iota(jnp.int32, (BE, 1), 0)
    valid = (rows < E).astype(jnp.float32)
    z = jnp.zeros((BE, TW - H - 1), jnp.float32)
    m_ref[...] = jnp.concatenate([mm * valid, valid, z], axis=1)


def _edge_mlp(g, W_e2, b2r, sel):
    return pl.pallas_call(
        _edge_body,
        grid=(E_P // BE,),
        in_specs=[
            pl.BlockSpec((BE, TW), lambda i: (i, 0)),
            pl.BlockSpec((H, H), lambda i: (0, 0)),
            pl.BlockSpec((1, H), lambda i: (0, 0)),
            pl.BlockSpec((2 * TW, H), lambda i: (0, 0)),
        ],
        out_specs=pl.BlockSpec((BE, TW), lambda i: (i, 0)),
        out_shape=jax.ShapeDtypeStruct((E_P, TW), jnp.float32),
    )(g, W_e2, b2r, sel)


N_P = 10240
_RPT = N_P // NS


@functools.cache
def _make_scatter_kernel():
    @functools.partial(
        pl.kernel,
        out_type=jax.ShapeDtypeStruct((NC, N_P, TW), jnp.float32),
        mesh=_sc_mesh(),
        scratch_types=[
            pltpu.VMEM((NCH, C), jnp.int32),
            pltpu.VMEM((C, TW), jnp.float32),
            pltpu.VMEM((C, TW), jnp.float32),
            pltpu.VMEM_SHARED((N_P, TW), jnp.float32),
            pltpu.SemaphoreType.DMA((2,)),
        ],
    )
    def _scatter_kernel(m_hbm, idxd_hbm, p_hbm, idx_v, mb0, mb1, acc, lsem):
        cid = lax.axis_index("c")
        sid = lax.axis_index("s")
        wid = sid * NC + cid
        pltpu.sync_copy(idxd_hbm.at[wid], idx_v)

        @pl.loop(0, C * TW // 16)
        def _zero(i):
            mb0[i // (TW // 16), pl.ds((i % (TW // 16)) * 16, 16)] = (
                jnp.zeros((16,), jnp.float32))

        for r in range(_RPT // C):
            pltpu.sync_copy(mb0, acc.at[pl.ds(sid * _RPT + r * C, C)])
        plsc.subcore_barrier()

        base = wid * EW
        mbufs = (mb0, mb1)
        for b in range(2):
            pltpu.async_copy(m_hbm.at[pl.ds(base + b * C, C)], mbufs[b],
                             lsem.at[b])

        @pl.loop(0, NCH // 2)
        def _grp(g):
            for b in range(2):
                j = 2 * g + b
                pltpu.make_async_copy(m_hbm.at[pl.ds(base, C)], mbufs[b],
                                      lsem.at[b]).wait()
                pltpu.sync_copy(mbufs[b], acc.at[idx_v.at[j]], add=True)
                nj = j + 2

                @pl.when(nj < NCH)
                def _():
                    pltpu.async_copy(m_hbm.at[pl.ds(base + nj * C, C)],
                                     mbufs[b], lsem.at[b])

        plsc.subcore_barrier()
        pltpu.sync_copy(acc.at[pl.ds(sid * _RPT, _RPT)],
                        p_hbm.at[cid, pl.ds(sid * _RPT, _RPT)])

    return _scatter_kernel


def _sc_scatter(m, dst_p):
    return _make_scatter_kernel()(m, dst_p)


def _node_body(x_ref, p_ref, wn1_ref, bn1_ref, wn2_ref, bn2_ref, o_ref):
    p0 = p_ref[0]
    p1 = p_ref[1]
    agg = p0[:, 0:H] + p1[:, 0:H]
    cnt = p0[:, H:H + 1] + p1[:, H:H + 1]
    mean = agg / jnp.maximum(cnt, 1.0)
    h1 = (jnp.dot(x_ref[...], wn1_ref[0:DIN, :],
                  preferred_element_type=jnp.float32)
          + jnp.dot(mean, wn1_ref[DIN:DIN + H, :],
                    preferred_element_type=jnp.float32)
          + bn1_ref[...])
    h1 = h1 * jax.nn.sigmoid(h1)
    o_ref[...] = (jnp.dot(h1, wn2_ref[...], preferred_element_type=jnp.float32)
                  + bn2_ref[...])


def _node_mlp(x, p, W_n1, bn1r, W_n2, bn2r):
    return pl.pallas_call(
        _node_body,
        grid=(N // BN,),
        in_specs=[
            pl.BlockSpec((BN, DIN), lambda i: (i, 0)),
            pl.BlockSpec((NC, BN, TW), lambda i: (0, i, 0)),
            pl.BlockSpec((DIN + H, H), lambda i: (0, 0)),
            pl.BlockSpec((1, H), lambda i: (0, 0)),
            pl.BlockSpec((H, DOUT), lambda i: (0, 0)),
            pl.BlockSpec((1, DOUT), lambda i: (0, 0)),
        ],
        out_specs=pl.BlockSpec((BN, DOUT), lambda i: (i, 0)),
        out_shape=jax.ShapeDtypeStruct((N, DOUT), jnp.float32),
    )(x, p, W_n1, bn1r, W_n2, bn2r)


def kernel(x, pos, edge_index, W_e1, b_e1, W_e2, b_e2, W_x, b_x,
           W_n1, b_n1, W_n2, b_n2):
    src = edge_index[0].astype(jnp.int32)
    dst = edge_index[1].astype(jnp.int32)
    dst_c = jnp.pad(dst, (0, E_P - E)).reshape(NCHUNK, C)
    src_c = jnp.pad(src, (0, E_P - E)).reshape(NCHUNK, C)
    dst_p = dst_c.reshape(NW, NCH, C)

    b1r = b_e1.reshape(1, H)
    b2r = b_e2.reshape(1, H)
    bn1r = b_n1.reshape(1, H)
    bn2r = b_n2.reshape(1, DOUT)
    sel = jnp.zeros((2 * TW, H), jnp.float32)
    sel = sel.at[0:H, :].set(jnp.eye(H, dtype=jnp.float32))
    sel = sel.at[TW + H:TW + H + 3, :].set(
        jnp.broadcast_to(W_e1[2 * DIN], (3, H)))

    td, ts = _build_tables(x, pos, W_e1, b1r)
    g = _sc_gather(td, ts, dst_c, src_c)
    m = _edge_mlp(g, W_e2, b2r, sel)
    p = _sc_scatter(m, dst_p)
    return _node_mlp(x, p, W_n1, bn1r, W_n2, bn2r)


# speedup vs baseline: 1.3600x; 1.3600x over previous
import functools

import jax
import jax.numpy as jnp
from jax import lax
from jax.experimental import pallas as pl
from jax.experimental.pallas import tpu as pltpu
from jax.experimental.pallas import tpu_sc as plsc

N = 10000
E = 320000
DIN = 128
H = 64
DOUT = 32
TW = 128

NC = 2
NS = 16
NW = NC * NS
C = 128
EW = 10240
NCH = EW // C
E_P = NW * EW

BN = 2000
BE = 2048

@functools.cache
def _sc_mesh():
    return plsc.VectorSubcoreMesh(
        core_axis_name="c", subcore_axis_name="s",
        num_cores=NC, num_subcores=NS)


def _tables_body(x_ref, pos_ref, w1_ref, b1_ref, td_ref, ts_ref):
    xb = x_ref[...]
    a = jnp.dot(xb, w1_ref[0:DIN, :], preferred_element_type=jnp.float32)
    b = jnp.dot(xb, w1_ref[DIN:2 * DIN, :], preferred_element_type=jnp.float32)
    p = pos_ref[...]
    z = jnp.zeros((BN, TW - H - 3), jnp.float32)
    td_ref[...] = jnp.concatenate([a + b1_ref[...], p, z], axis=1)
    ts_ref[...] = jnp.concatenate([b, -p, z], axis=1)


def _build_tables(x, pos, W_e1, b1r):
    return pl.pallas_call(
        _tables_body,
        grid=(N // BN,),
        in_specs=[
            pl.BlockSpec((BN, DIN), lambda i: (i, 0)),
            pl.BlockSpec((BN, 3), lambda i: (i, 0)),
            pl.BlockSpec((2 * DIN + 1, H), lambda i: (0, 0)),
            pl.BlockSpec((1, H), lambda i: (0, 0)),
        ],
        out_specs=[
            pl.BlockSpec((BN, TW), lambda i: (i, 0)),
            pl.BlockSpec((BN, TW), lambda i: (i, 0)),
        ],
        out_shape=[
            jax.ShapeDtypeStruct((N, TW), jnp.float32),
            jax.ShapeDtypeStruct((N, TW), jnp.float32),
        ],
    )(x, pos, W_e1, b1r)


NBUF = 4
NCHUNK = E_P // C
NHALF = 2
CH_H = NCHUNK // NHALF
K_H = CH_H // NW
E_H = E_P // NHALF


@functools.cache
def _make_gather_kernel():
    @functools.partial(
        pl.kernel,
        out_type=jax.ShapeDtypeStruct((E_H, TW), jnp.float32),
        mesh=_sc_mesh(),
        scratch_types=[
            pltpu.VMEM((K_H, C), jnp.int32),
            pltpu.VMEM((K_H, C), jnp.int32),
        ] + [pltpu.VMEM((C, TW), jnp.float32) for _ in range(NBUF)] + [
            pltpu.SemaphoreType.DMA((NBUF,)),
            pltpu.SemaphoreType.DMA((NBUF,)),
        ],
    )
    def _gather_kernel(td_hbm, ts_hbm, idxd_hbm, idxs_hbm, g_hbm,
                       idxd_v, idxs_v, b0, b1, b2, b3, gsem, osem):
        wid = lax.axis_index("s") * NC + lax.axis_index("c")
        cbase = wid * K_H
        pltpu.sync_copy(idxd_hbm.at[pl.ds(cbase, K_H)], idxd_v)
        pltpu.sync_copy(idxs_hbm.at[pl.ds(cbase, K_H)], idxs_v)
        bufs = (b0, b1, b2, b3)

        for b in range(NBUF):
            pltpu.async_copy(td_hbm.at[idxd_v.at[b]], bufs[b], gsem.at[b])

        @pl.loop(0, K_H // NBUF)
        def _grp(g):
            for b in range(NBUF):
                pltpu.make_async_copy(
                    td_hbm.at[idxd_v.at[b]], bufs[b], gsem.at[b]).wait()
                pltpu.async_copy(ts_hbm.at[idxs_v.at[g * NBUF + b]], bufs[b],
                                 gsem.at[b], add=True)
            for b in range(NBUF):
                j = g * NBUF + b
                pltpu.make_async_copy(
                    ts_hbm.at[idxs_v.at[b]], bufs[b], gsem.at[b]).wait()
                pltpu.async_copy(bufs[b],
                                 g_hbm.at[pl.ds((cbase + j) * C, C)],
                                 osem.at[b])
            for b in range(NBUF):
                j = (g + 1) * NBUF + b

                @pl.when(j < K_H)
                def _():
                    pltpu.make_async_copy(
                        bufs[b], g_hbm.at[pl.ds(0, C)], osem.at[b]).wait()
                    pltpu.async_copy(td_hbm.at[idxd_v.at[j]], bufs[b],
                                     gsem.at[b])

        for b in range(NBUF):
            pltpu.make_async_copy(
                bufs[b], g_hbm.at[pl.ds(0, C)], osem.at[b]).wait()

    return _gather_kernel


def _sc_gather(td, ts, dst_h, src_h):
    return _make_gather_kernel()(td, ts, dst_h, src_h)


def _make_edge_body(row0):
    def _edge_body(g_ref, w2_ref, b2_ref, sel_ref, m_ref):
        i = pl.program_id(0)
        g = g_ref[...]
        pre = (jnp.dot(g, sel_ref[0:TW, :],
                       preferred_element_type=jnp.float32)
               + jnp.dot(g * g, sel_ref[TW:2 * TW, :],
                         preferred_element_type=jnp.float32))
        h1 = pre * jax.nn.sigmoid(pre)
        mm = (jnp.dot(h1, w2_ref[...], preferred_element_type=jnp.float32)
              + b2_ref[...])
        mm = mm * jax.nn.sigmoid(mm)
        rows = row0 + i * BE + lax.broadcasted_iota(jnp.int32, (BE, 1), 0)
        valid = (rows < E).astype(jnp.float32)
        z = jnp.zeros((BE, TW - H - 1), jnp.float32)
        m_ref[...] = jnp.concatenate([mm * valid, valid, z], axis=1)

    return _edge_body


def _edge_mlp(g, W_e2, b2r, sel, row0):
    return pl.pallas_call(
        _make_edge_body(row0),
        grid=(E_H // BE,),
        in_specs=[
            pl.BlockSpec((BE, TW), lambda i: (i, 0)),
            pl.BlockSpec((H, H), lambda i: (0, 0)),
            pl.BlockSpec((1, H), lambda i: (0, 0)),
            pl.BlockSpec((2 * TW, H), lambda i: (0, 0)),
        ],
        out_specs=pl.BlockSpec((BE, TW), lambda i: (i, 0)),
        out_shape=jax.ShapeDtypeStruct((E_H, TW), jnp.float32),
    )(g, W_e2, b2r, sel)


N_P = 10240
_RPT = N_P // NS


@functools.cache
def _make_scatter_kernel():
    @functools.partial(
        pl.kernel,
        out_type=jax.ShapeDtypeStruct((NC, N_P, TW), jnp.float32),
        mesh=_sc_mesh(),
        scratch_types=[
            pltpu.VMEM((K_H, C), jnp.int32),
            pltpu.VMEM((C, TW), jnp.float32),
            pltpu.VMEM((C, TW), jnp.float32),
            pltpu.VMEM_SHARED((N_P, TW), jnp.float32),
            pltpu.SemaphoreType.DMA((2,)),
        ],
    )
    def _scatter_kernel(m_hbm, idxd_hbm, p_hbm, idx_v, mb0, mb1, acc, lsem):
        cid = lax.axis_index("c")
        sid = lax.axis_index("s")
        wid = sid * NC + cid
        pltpu.sync_copy(idxd_hbm.at[wid], idx_v)

        @pl.loop(0, C * TW // 16)
        def _zero(i):
            mb0[i // (TW // 16), pl.ds((i % (TW // 16)) * 16, 16)] = (
                jnp.zeros((16,), jnp.float32))

        for r in range(_RPT // C):
            pltpu.sync_copy(mb0, acc.at[pl.ds(sid * _RPT + r * C, C)])
        plsc.subcore_barrier()

        base = wid * K_H * C
        mbufs = (mb0, mb1)
        for b in range(2):
            pltpu.async_copy(m_hbm.at[pl.ds(base + b * C, C)], mbufs[b],
                             lsem.at[b])

        @pl.loop(0, K_H // 2)
        def _grp(g):
            for b in range(2):
                j = 2 * g + b
                pltpu.make_async_copy(m_hbm.at[pl.ds(base, C)], mbufs[b],
                                      lsem.at[b]).wait()
                pltpu.sync_copy(mbufs[b], acc.at[idx_v.at[j]], add=True)
                nj = j + 2

                @pl.when(nj < K_H)
                def _():
                    pltpu.async_copy(m_hbm.at[pl.ds(base + nj * C, C)],
                                     mbufs[b], lsem.at[b])

        plsc.subcore_barrier()
        pltpu.sync_copy(acc.at[pl.ds(sid * _RPT, _RPT)],
                        p_hbm.at[cid, pl.ds(sid * _RPT, _RPT)])

    return _scatter_kernel


def _sc_scatter(m, dst_p):
    return _make_scatter_kernel()(m, dst_p)


def _node_body(x_ref, pa_ref, pb_ref, wn1_ref, bn1_ref, wn2_ref, bn2_ref,
               o_ref):
    p0 = pa_ref[0] + pb_ref[0]
    p1 = pa_ref[1] + pb_ref[1]
    agg = p0[:, 0:H] + p1[:, 0:H]
    cnt = p0[:, H:H + 1] + p1[:, H:H + 1]
    mean = agg / jnp.maximum(cnt, 1.0)
    h1 = (jnp.dot(x_ref[...], wn1_ref[0:DIN, :],
                  preferred_element_type=jnp.float32)
          + jnp.dot(mean, wn1_ref[DIN:DIN + H, :],
                    preferred_element_type=jnp.float32)
          + bn1_ref[...])
    h1 = h1 * jax.nn.sigmoid(h1)
    o_ref[...] = (jnp.dot(h1, wn2_ref[...], preferred_element_type=jnp.float32)
                  + bn2_ref[...])


def _node_mlp(x, pa, pb, W_n1, bn1r, W_n2, bn2r):
    return pl.pallas_call(
        _node_body,
        grid=(N // BN,),
        in_specs=[
            pl.BlockSpec((BN, DIN), lambda i: (i, 0)),
            pl.BlockSpec((NC, BN, TW), lambda i: (0, i, 0)),
            pl.BlockSpec((NC, BN, TW), lambda i: (0, i, 0)),
            pl.BlockSpec((DIN + H, H), lambda i: (0, 0)),
            pl.BlockSpec((1, H), lambda i: (0, 0)),
            pl.BlockSpec((H, DOUT), lambda i: (0, 0)),
            pl.BlockSpec((1, DOUT), lambda i: (0, 0)),
        ],
        out_specs=pl.BlockSpec((BN, DOUT), lambda i: (i, 0)),
        out_shape=jax.ShapeDtypeStruct((N, DOUT), jnp.float32),
    )(x, pa, pb, W_n1, bn1r, W_n2, bn2r)


def kernel(x, pos, edge_index, W_e1, b_e1, W_e2, b_e2, W_x, b_x,
           W_n1, b_n1, W_n2, b_n2):
    src = edge_index[0].astype(jnp.int32)
    dst = edge_index[1].astype(jnp.int32)
    dst_c = jnp.pad(dst, (0, E_P - E)).reshape(NCHUNK, C)
    src_c = jnp.pad(src, (0, E_P - E)).reshape(NCHUNK, C)

    b1r = b_e1.reshape(1, H)
    b2r = b_e2.reshape(1, H)
    bn1r = b_n1.reshape(1, H)
    bn2r = b_n2.reshape(1, DOUT)
    sel = jnp.zeros((2 * TW, H), jnp.float32)
    sel = sel.at[0:H, :].set(jnp.eye(H, dtype=jnp.float32))
    sel = sel.at[TW + H:TW + H + 3, :].set(
        jnp.broadcast_to(W_e1[2 * DIN], (3, H)))

    td, ts = _build_tables(x, pos, W_e1, b1r)
    parts = []
    for h in range(NHALF):
        ch0 = h * CH_H
        dst_h = dst_c[ch0:ch0 + CH_H]
        src_h = src_c[ch0:ch0 + CH_H]
        g = _sc_gather(td, ts, dst_h, src_h)
        m = _edge_mlp(g, W_e2, b2r, sel, h * E_H)
        parts.append(_sc_scatter(m, dst_h.reshape(NW, K_H, C)))
    return _node_mlp(x, parts[0], parts[1], W_n1, bn1r, W_n2, bn2r)


# speedup vs baseline: 1.4318x; 1.0528x over previous
import functools

import jax
import jax.numpy as jnp
from jax import lax
from jax.experimental import pallas as pl
from jax.experimental.pallas import tpu as pltpu
from jax.experimental.pallas import tpu_sc as plsc

N = 10000
E = 320000
DIN = 128
H = 64
DOUT = 32
TW = 128

NC = 2
NS = 16
NW = NC * NS
C = 128
EW = 10240
NCH = EW // C
E_P = NW * EW

BN = 2000
BE = 2048

@functools.cache
def _sc_mesh():
    return plsc.VectorSubcoreMesh(
        core_axis_name="c", subcore_axis_name="s",
        num_cores=NC, num_subcores=NS)


def _tables_body(x_ref, pos_ref, w1_ref, b1_ref, td_ref, ts_ref):
    xb = x_ref[...]
    a = jnp.dot(xb, w1_ref[0:DIN, :], preferred_element_type=jnp.float32)
    b = jnp.dot(xb, w1_ref[DIN:2 * DIN, :], preferred_element_type=jnp.float32)
    p = pos_ref[...]
    z = jnp.zeros((BN, TW - H - 3), jnp.float32)
    td_ref[...] = jnp.concatenate([a + b1_ref[...], p, z], axis=1)
    ts_ref[...] = jnp.concatenate([b, -p, z], axis=1)


def _build_tables(x, pos, W_e1, b1r):
    return pl.pallas_call(
        _tables_body,
        grid=(N // BN,),
        in_specs=[
            pl.BlockSpec((BN, DIN), lambda i: (i, 0)),
            pl.BlockSpec((BN, 3), lambda i: (i, 0)),
            pl.BlockSpec((2 * DIN + 1, H), lambda i: (0, 0)),
            pl.BlockSpec((1, H), lambda i: (0, 0)),
        ],
        out_specs=[
            pl.BlockSpec((BN, TW), lambda i: (i, 0)),
            pl.BlockSpec((BN, TW), lambda i: (i, 0)),
        ],
        out_shape=[
            jax.ShapeDtypeStruct((N, TW), jnp.float32),
            jax.ShapeDtypeStruct((N, TW), jnp.float32),
        ],
    )(x, pos, W_e1, b1r)


NBUF = 5
NCHUNK = E_P // C
NHALF = 2
CH_H = NCHUNK // NHALF
K_H = CH_H // NW
E_H = E_P // NHALF


@functools.cache
def _make_gather_kernel():
    @functools.partial(
        pl.kernel,
        out_type=jax.ShapeDtypeStruct((E_H, TW), jnp.float32),
        mesh=_sc_mesh(),
        scratch_types=[
            pltpu.VMEM((K_H, C), jnp.int32),
            pltpu.VMEM((K_H, C), jnp.int32),
        ] + [pltpu.VMEM((C, TW), jnp.float32) for _ in range(NBUF)] + [
            pltpu.SemaphoreType.DMA((NBUF,)),
            pltpu.SemaphoreType.DMA((NBUF,)),
        ],
    )
    def _gather_kernel(td_hbm, ts_hbm, idxd_hbm, idxs_hbm, g_hbm,
                       idxd_v, idxs_v, b0, b1, b2, b3, b4, gsem, osem):
        wid = lax.axis_index("s") * NC + lax.axis_index("c")
        cbase = wid * K_H
        pltpu.sync_copy(idxd_hbm.at[pl.ds(cbase, K_H)], idxd_v)
        pltpu.sync_copy(idxs_hbm.at[pl.ds(cbase, K_H)], idxs_v)
        bufs = (b0, b1, b2, b3, b4)

        for b in range(NBUF):
            pltpu.async_copy(td_hbm.at[idxd_v.at[b]], bufs[b], gsem.at[b])

        @pl.loop(0, K_H // NBUF)
        def _grp(g):
            for b in range(NBUF):
                pltpu.make_async_copy(
                    td_hbm.at[idxd_v.at[b]], bufs[b], gsem.at[b]).wait()
                pltpu.async_copy(ts_hbm.at[idxs_v.at[g * NBUF + b]], bufs[b],
                                 gsem.at[b], add=True)
            for b in range(NBUF):
                j = g * NBUF + b
                pltpu.make_async_copy(
                    ts_hbm.at[idxs_v.at[b]], bufs[b], gsem.at[b]).wait()
                pltpu.async_copy(bufs[b],
                                 g_hbm.at[pl.ds((cbase + j) * C, C)],
                                 osem.at[b])
            for b in range(NBUF):
                j = (g + 1) * NBUF + b

                @pl.when(j < K_H)
                def _():
                    pltpu.make_async_copy(
                        bufs[b], g_hbm.at[pl.ds(0, C)], osem.at[b]).wait()
                    pltpu.async_copy(td_hbm.at[idxd_v.at[j]], bufs[b],
                                     gsem.at[b])

        for b in range(NBUF):
            pltpu.make_async_copy(
                bufs[b], g_hbm.at[pl.ds(0, C)], osem.at[b]).wait()

    return _gather_kernel


def _sc_gather(td, ts, dst_h, src_h):
    return _make_gather_kernel()(td, ts, dst_h, src_h)


def _make_edge_body(row0):
    def _edge_body(g_ref, w2_ref, b2_ref, sel_ref, m_ref):
        i = pl.program_id(0)
        g = g_ref[...]
        pre = (jnp.dot(g, sel_ref[0:TW, :],
                       preferred_element_type=jnp.float32)
               + jnp.dot(g * g, sel_ref[TW:2 * TW, :],
                         preferred_element_type=jnp.float32))
        h1 = pre * jax.nn.sigmoid(pre)
        mm = (jnp.dot(h1, w2_ref[...], preferred_element_type=jnp.float32)
              + b2_ref[...])
        mm = mm * jax.nn.sigmoid(mm)
        rows = row0 + i * BE + lax.broadcasted_iota(jnp.int32, (BE, 1), 0)
        valid = (rows < E).astype(jnp.float32)
        z = jnp.zeros((BE, TW - H - 1), jnp.float32)
        m_ref[...] = jnp.concatenate([mm * valid, valid, z], axis=1)

    return _edge_body


def _edge_mlp(g, W_e2, b2r, sel, row0):
    return pl.pallas_call(
        _make_edge_body(row0),
        grid=(E_H // BE,),
        in_specs=[
            pl.BlockSpec((BE, TW), lambda i: (i, 0)),
            pl.BlockSpec((H, H), lambda i: (0, 0)),
            pl.BlockSpec((1, H), lambda i: (0, 0)),
            pl.BlockSpec((2 * TW, H), lambda i: (0, 0)),
        ],
        out_specs=pl.BlockSpec((BE, TW), lambda i: (i, 0)),
        out_shape=jax.ShapeDtypeStruct((E_H, TW), jnp.float32),
    )(g, W_e2, b2r, sel)


N_P = 10240
_RPT = N_P // NS


@functools.cache
def _make_scatter_kernel():
    @functools.partial(
        pl.kernel,
        out_type=jax.ShapeDtypeStruct((NC, N_P, TW), jnp.float32),
        mesh=_sc_mesh(),
        scratch_types=[
            pltpu.VMEM((K_H, C), jnp.int32),
            pltpu.VMEM((C, TW), jnp.float32),
            pltpu.VMEM((C, TW), jnp.float32),
            pltpu.VMEM_SHARED((N_P, TW), jnp.float32),
            pltpu.SemaphoreType.DMA((2,)),
        ],
    )
    def _scatter_kernel(m_hbm, idxd_hbm, p_hbm, idx_v, mb0, mb1, acc, lsem):
        cid = lax.axis_index("c")
        sid = lax.axis_index("s")
        wid = sid * NC + cid
        pltpu.sync_copy(idxd_hbm.at[wid], idx_v)

        @pl.loop(0, C * TW // 16)
        def _zero(i):
            mb0[i // (TW // 16), pl.ds((i % (TW // 16)) * 16, 16)] = (
                jnp.zeros((16,), jnp.float32))

        for r in range(_RPT // C):
            pltpu.sync_copy(mb0, acc.at[pl.ds(sid * _RPT + r * C, C)])
        plsc.subcore_barrier()

        base = wid * K_H * C
        mbufs = (mb0, mb1)
        for b in range(2):
            pltpu.async_copy(m_hbm.at[pl.ds(base + b * C, C)], mbufs[b],
                             lsem.at[b])

        @pl.loop(0, K_H // 2)
        def _grp(g):
            for b in range(2):
                j = 2 * g + b
                pltpu.make_async_copy(m_hbm.at[pl.ds(base, C)], mbufs[b],
                                      lsem.at[b]).wait()
                pltpu.sync_copy(mbufs[b], acc.at[idx_v.at[j]], add=True)
                nj = j + 2

                @pl.when(nj < K_H)
                def _():
                    pltpu.async_copy(m_hbm.at[pl.ds(base + nj * C, C)],
                                     mbufs[b], lsem.at[b])

        plsc.subcore_barrier()
        pltpu.sync_copy(acc.at[pl.ds(sid * _RPT, _RPT)],
                        p_hbm.at[cid, pl.ds(sid * _RPT, _RPT)])

    return _scatter_kernel


def _sc_scatter(m, dst_p):
    return _make_scatter_kernel()(m, dst_p)


def _node_body(x_ref, pa_ref, pb_ref, wn1_ref, bn1_ref, wn2_ref, bn2_ref,
               o_ref):
    p0 = pa_ref[0] + pb_ref[0]
    p1 = pa_ref[1] + pb_ref[1]
    agg = p0[:, 0:H] + p1[:, 0:H]
    cnt = p0[:, H:H + 1] + p1[:, H:H + 1]
    mean = agg / jnp.maximum(cnt, 1.0)
    h1 = (jnp.dot(x_ref[...], wn1_ref[0:DIN, :],
                  preferred_element_type=jnp.float32)
          + jnp.dot(mean, wn1_ref[DIN:DIN + H, :],
                    preferred_element_type=jnp.float32)
          + bn1_ref[...])
    h1 = h1 * jax.nn.sigmoid(h1)
    o_ref[...] = (jnp.dot(h1, wn2_ref[...], preferred_element_type=jnp.float32)
                  + bn2_ref[...])


def _node_mlp(x, pa, pb, W_n1, bn1r, W_n2, bn2r):
    return pl.pallas_call(
        _node_body,
        grid=(N // BN,),
        in_specs=[
            pl.BlockSpec((BN, DIN), lambda i: (i, 0)),
            pl.BlockSpec((NC, BN, TW), lambda i: (0, i, 0)),
            pl.BlockSpec((NC, BN, TW), lambda i: (0, i, 0)),
            pl.BlockSpec((DIN + H, H), lambda i: (0, 0)),
            pl.BlockSpec((1, H), lambda i: (0, 0)),
            pl.BlockSpec((H, DOUT), lambda i: (0, 0)),
            pl.BlockSpec((1, DOUT), lambda i: (0, 0)),
        ],
        out_specs=pl.BlockSpec((BN, DOUT), lambda i: (i, 0)),
        out_shape=jax.ShapeDtypeStruct((N, DOUT), jnp.float32),
    )(x, pa, pb, W_n1, bn1r, W_n2, bn2r)


def kernel(x, pos, edge_index, W_e1, b_e1, W_e2, b_e2, W_x, b_x,
           W_n1, b_n1, W_n2, b_n2):
    src = edge_index[0].astype(jnp.int32)
    dst = edge_index[1].astype(jnp.int32)
    dst_c = jnp.pad(dst, (0, E_P - E)).reshape(NCHUNK, C)
    src_c = jnp.pad(src, (0, E_P - E)).reshape(NCHUNK, C)

    b1r = b_e1.reshape(1, H)
    b2r = b_e2.reshape(1, H)
    bn1r = b_n1.reshape(1, H)
    bn2r = b_n2.reshape(1, DOUT)
    sel = jnp.zeros((2 * TW, H), jnp.float32)
    sel = sel.at[0:H, :].set(jnp.eye(H, dtype=jnp.float32))
    sel = sel.at[TW + H:TW + H + 3, :].set(
        jnp.broadcast_to(W_e1[2 * DIN], (3, H)))

    td, ts = _build_tables(x, pos, W_e1, b1r)
    parts = []
    for h in range(NHALF):
        ch0 = h * CH_H
        dst_h = dst_c[ch0:ch0 + CH_H]
        src_h = src_c[ch0:ch0 + CH_H]
        g = _sc_gather(td, ts, dst_h, src_h)
        m = _edge_mlp(g, W_e2, b2r, sel, h * E_H)
        parts.append(_sc_scatter(m, dst_h.reshape(NW, K_H, C)))
    return _node_mlp(x, parts[0], parts[1], W_n1, bn1r, W_n2, bn2r)
